# Initial kernel scaffold; baseline (speedup 1.0000x reference)
import jax
import jax.numpy as jnp
from jax.experimental import pallas as pl


def kernel(x_val, w0, w, v, x_idx):
    raise NotImplementedError("write your pallas kernel here")


# speedup vs baseline: 1.2117x; 1.2117x over previous
import dataclasses

import jax
import jax.numpy as jnp
from jax import lax
from jax.experimental import pallas as pl
from jax.experimental.pallas import tpu as pltpu
from jax.experimental.pallas import tpu_sc as plsc

_B, _F = 16384, 26
_V, _D = 1000000, 16
_NC, _NS = 2, 16
_NW = _NC * _NS
_RPW = _B // _NW
_C = 64
_NCH = _RPW // _C
_K = _C * _F
_KG = 128
_NG = _K // _KG


def _fm_body(x_val_hbm, w0_hbm, w_hbm, v_hbm, x_idx_hbm, out_hbm,
             idx_v, val_v, rows_v, wg_v, out_v, w0_v, sem):
    wid = lax.axis_index("s") * _NC + lax.axis_index("c")
    pltpu.sync_copy(w0_hbm, w0_v)
    w0s = w0_v[...][0]
    lane = lax.iota(jnp.int32, 16)
    ones = jnp.full((16,), 1.0, jnp.float32)
    zeros16 = jnp.zeros((16,), jnp.float32)
    tail_mask = jnp.where(lane < _F - 16, ones, zeros16)

    @pl.loop(0, _NCH)
    def _chunk(c):
        row0 = wid * _RPW + c * _C
        i0 = row0 * _F
        pltpu.sync_copy(x_idx_hbm.at[pl.ds(i0, _K)], idx_v)
        pltpu.sync_copy(x_val_hbm.at[pl.ds(i0, _K)], val_v.at[pl.ds(0, _K)])
        copies = []
        for j in range(_NG):
            copies.append(pltpu.async_copy(
                v_hbm.at[idx_v.at[pl.ds(j * _KG, _KG)]], rows_v.at[pl.ds(j * _KG, _KG)], sem))
        for j in range(_NG):
            copies.append(pltpu.async_copy(
                w_hbm.at[idx_v.at[pl.ds(j * _KG, _KG)]], wg_v.at[pl.ds(j * _KG, _KG)], sem))
        for cp in copies:
            cp.wait()

        @pl.loop(0, _C // 16)
        def _group(g):
            res = zeros16
            for l in range(16):
                o = (g * 16 + l) * _F
                va = val_v[pl.ds(o, 16)]
                vb = val_v[pl.ds(o + 16, 16)]
                wa = wg_v[pl.ds(o, 16)]
                wb = wg_v[pl.ds(o + 16, 16)]
                lin = w0s + jnp.sum(va * wa) + jnp.sum(vb * wb * tail_mask)
                xv = zeros16
                xsq = zeros16
                for f in range(_F):
                    s = va[f] if f < 16 else vb[f - 16]
                    r = rows_v[o + f, :]
                    p = s * r
                    xv = xv + p
                    xsq = xsq + p * r
                tot = lin + 0.5 * jnp.sum(xv * xv - xsq)
                res = jnp.where(lane == l, tot, res)
            out_v[pl.ds(g * 16, 16)] = res

        pltpu.sync_copy(out_v, out_hbm.at[pl.ds(row0, _C)])


def kernel(x_val, w0, w, v, x_idx):
    idx_flat = x_idx.astype(jnp.int32).reshape(_B * _F)
    val_flat = x_val.reshape(_B * _F)
    mesh = plsc.VectorSubcoreMesh(core_axis_name="c", subcore_axis_name="s")
    cp = pltpu.CompilerParams()
    if "needs_layout_passes" in pltpu.CompilerParams.__dataclass_fields__:
        cp = dataclasses.replace(cp, needs_layout_passes=False)
    if "use_tc_tiling_on_sc" in pltpu.CompilerParams.__dataclass_fields__:
        cp = dataclasses.replace(cp, use_tc_tiling_on_sc=False)
    fm = pl.kernel(
        _fm_body,
        out_type=jax.ShapeDtypeStruct((_B,), jnp.float32),
        mesh=mesh,
        compiler_params=cp,
        scratch_types=[
            pltpu.VMEM((_K,), jnp.int32),
            pltpu.VMEM((_K + 16,), jnp.float32),
            pltpu.VMEM((_K, _D), jnp.float32),
            pltpu.VMEM((_K + 16,), jnp.float32),
            pltpu.VMEM((_C,), jnp.float32),
            pltpu.VMEM((16,), jnp.float32),
            pltpu.SemaphoreType.DMA,
        ],
    )
    return fm(val_flat, jnp.tile(w0, 16), w, v, idx_flat)
